# Initial kernel scaffold; baseline (speedup 1.0000x reference)
import jax
import jax.numpy as jnp
from jax.experimental import pallas as pl


def kernel(x, edge_index, edge_embed, W1, b1, gamma, beta, W2, b2, eps):
    raise NotImplementedError("write your pallas kernel here")


# speedup vs baseline: 4.0149x; 4.0149x over previous
import functools

import jax
import jax.numpy as jnp
from jax import lax
from jax.experimental import pallas as pl
from jax.experimental.pallas import tpu as pltpu
from jax.experimental.pallas import tpu_sc as plsc

N = 10000
E = 320000
D = 128

NC = 2
NS = 16
NW = NC * NS
EPW = E // NW
K = 80
NBLK = EPW // K
RPT = 624
TAIL = N - NS * RPT

_mesh = plsc.VectorSubcoreMesh(core_axis_name="c", subcore_axis_name="s")


@functools.partial(
    pl.kernel,
    mesh=_mesh,
    out_type=jax.ShapeDtypeStruct((NC, N, D), jnp.float32),
    scratch_types=[
        pltpu.VMEM_SHARED((N, D), jnp.float32),
        pltpu.VMEM((K,), jnp.int32),
        pltpu.VMEM((K,), jnp.int32),
        pltpu.VMEM((K, D), jnp.float32),
        pltpu.VMEM((K, D), jnp.float32),
        pltpu.SemaphoreType.DMA,
    ],
)
def _sc_agg(x_hbm, row_hbm, col_hbm, ee_hbm, zero_hbm, out_hbm,
            agg_sh, colv, rowv, xg, eev, sem):
    c = lax.axis_index("c")
    s = lax.axis_index("s")
    wid = c * NS + s

    pltpu.sync_copy(zero_hbm.at[pl.ds(s * RPT, RPT)],
                    agg_sh.at[pl.ds(s * RPT, RPT)])

    @pl.when(s == NS - 1)
    def _():
        pltpu.sync_copy(zero_hbm.at[pl.ds(NS * RPT, TAIL)],
                        agg_sh.at[pl.ds(NS * RPT, TAIL)])

    plsc.subcore_barrier()

    base = wid * EPW

    def block(b, carry):
        off = base + b * K
        pltpu.sync_copy(col_hbm.at[pl.ds(off, K)], colv)
        pltpu.sync_copy(row_hbm.at[pl.ds(off, K)], rowv)
        gather = pltpu.async_copy(x_hbm.at[colv], xg, sem)
        pltpu.sync_copy(ee_hbm.at[pl.ds(off, K)], eev)
        gather.wait()

        def mrow(r, carry2):
            for j in range(D // 16):
                sl = pl.ds(j * 16, 16)
                xg[r, sl] = xg[r, sl] * eev[r, sl]
            return carry2

        lax.fori_loop(0, K, mrow, 0)
        pltpu.sync_copy(xg, agg_sh.at[rowv], add=True)
        return carry

    lax.fori_loop(0, NBLK, block, 0)
    plsc.subcore_barrier()

    pltpu.sync_copy(agg_sh.at[pl.ds(s * RPT, RPT)],
                    out_hbm.at[c, pl.ds(s * RPT, RPT)])

    @pl.when(s == NS - 1)
    def _():
        pltpu.sync_copy(agg_sh.at[pl.ds(NS * RPT, TAIL)],
                        out_hbm.at[c, pl.ds(NS * RPT, TAIL)])


def _mlp_body(x_ref, p_ref, w1t_ref, b1_ref, gamma_ref, beta_ref,
              w2t_ref, b2_ref, scale_ref, out_ref):
    h = scale_ref[0, 0] * x_ref[...] + p_ref[0] + p_ref[1]
    h1 = jnp.dot(h, w1t_ref[...], preferred_element_type=jnp.float32)
    h1 = h1 + b1_ref[...]
    mean = jnp.mean(h1, axis=0, keepdims=True)
    cent = h1 - mean
    var = jnp.mean(cent * cent, axis=0, keepdims=True)
    hn = cent * lax.rsqrt(var + 1e-5) * gamma_ref[...] + beta_ref[...]
    hr = jnp.maximum(hn, 0.0)
    out = jnp.dot(hr, w2t_ref[...], preferred_element_type=jnp.float32)
    out_ref[...] = out + b2_ref[...]


_mlp_call = pl.pallas_call(
    _mlp_body,
    out_shape=jax.ShapeDtypeStruct((N, D), jnp.float32),
)


def kernel(x, edge_index, edge_embed, W1, b1, gamma, beta, W2, b2, eps):
    row = edge_index[0].astype(jnp.int32)
    col = edge_index[1].astype(jnp.int32)
    zero = jnp.zeros((N, D), jnp.float32)
    partials = _sc_agg(x, row, col, edge_embed, zero)
    scale = (1.0 + eps[0]).reshape(1, 1)
    return _mlp_call(x, partials, W1.T, b1.reshape(1, D),
                     gamma.reshape(1, D), beta.reshape(1, D),
                     W2.T, b2.reshape(1, D), scale)


# speedup vs baseline: 8.0112x; 1.9954x over previous
import functools

import jax
import jax.numpy as jnp
from jax import lax
from jax.experimental import pallas as pl
from jax.experimental.pallas import tpu as pltpu
from jax.experimental.pallas import tpu_sc as plsc

N = 10000
E = 320000
D = 128

NC = 2
NS = 16
NW = NC * NS
EPW = E // NW
K = 40
NBLK = EPW // K
NBUF = 4
RPT = 624
TAIL = N - NS * RPT

_mesh = plsc.VectorSubcoreMesh(core_axis_name="c", subcore_axis_name="s")


@functools.partial(
    pl.kernel,
    mesh=_mesh,
    out_type=jax.ShapeDtypeStruct((NC, N, D), jnp.float32),
    scratch_types=(
        [pltpu.VMEM_SHARED((N, D), jnp.float32)]
        + [pltpu.VMEM((K,), jnp.int32)] * NBUF
        + [pltpu.VMEM((K,), jnp.int32)] * NBUF
        + [pltpu.VMEM((K, D), jnp.float32)] * NBUF
        + [pltpu.VMEM((K, D), jnp.float32)] * NBUF
        + [pltpu.SemaphoreType.DMA] * (5 * NBUF)
    ),
)
def _sc_agg(x_hbm, row_hbm, col_hbm, ee_hbm, zero_hbm, out_hbm,
            agg_sh, *bufs):
    colv = bufs[0:NBUF]
    rowv = bufs[NBUF:2 * NBUF]
    xg = bufs[2 * NBUF:3 * NBUF]
    eev = bufs[3 * NBUF:4 * NBUF]
    csem = bufs[4 * NBUF:5 * NBUF]
    rsem = bufs[5 * NBUF:6 * NBUF]
    gsem = bufs[6 * NBUF:7 * NBUF]
    esem = bufs[7 * NBUF:8 * NBUF]
    ssem = bufs[8 * NBUF:9 * NBUF]

    c = lax.axis_index("c")
    s = lax.axis_index("s")
    wid = c * NS + s
    base = wid * EPW

    pltpu.sync_copy(zero_hbm.at[pl.ds(s * RPT, RPT)],
                    agg_sh.at[pl.ds(s * RPT, RPT)])

    @pl.when(s == NS - 1)
    def _():
        pltpu.sync_copy(zero_hbm.at[pl.ds(NS * RPT, TAIL)],
                        agg_sh.at[pl.ds(NS * RPT, TAIL)])

    plsc.subcore_barrier()

    def start_idx(n, q):
        off = base + n * K
        pltpu.async_copy(col_hbm.at[pl.ds(off, K)], colv[q], csem[q])
        pltpu.async_copy(row_hbm.at[pl.ds(off, K)], rowv[q], rsem[q])

    def start_fetch(n, q):
        pltpu.make_async_copy(col_hbm.at[pl.ds(0, K)], colv[q],
                              csem[q]).wait()
        pltpu.async_copy(x_hbm.at[colv[q]], xg[q], gsem[q])
        pltpu.async_copy(ee_hbm.at[pl.ds(base + n * K, K)], eev[q], esem[q])

    def wait_scatter(q):
        pltpu.make_async_copy(xg[q], agg_sh.at[rowv[q]], ssem[q]).wait()

    def compute(b, p):
        pltpu.make_async_copy(x_hbm.at[colv[p]], xg[p], gsem[p]).wait()
        pltpu.make_async_copy(ee_hbm.at[pl.ds(0, K)], eev[p], esem[p]).wait()

        def mrows(i, carry):
            r = i * 4
            for rr in range(4):
                for j in range(D // 16):
                    sl = pl.ds(j * 16, 16)
                    xg[p][r + rr, sl] = xg[p][r + rr, sl] * eev[p][r + rr, sl]
            return carry

        lax.fori_loop(0, K // 4, mrows, 0)
        pltpu.make_async_copy(row_hbm.at[pl.ds(0, K)], rowv[p],
                              rsem[p]).wait()
        pltpu.async_copy(xg[p], agg_sh.at[rowv[p]], ssem[p], add=True)

    for n in range(NBUF - 1):
        start_idx(n, n)
    for n in range(NBUF - 2):
        start_fetch(n, n)

    def outer(g, carry):
        for p in range(NBUF):
            b = g * NBUF + p
            compute(b, p)
            q3 = (p + NBUF - 1) % NBUF

            @pl.when(b >= 1)
            def _():
                wait_scatter(q3)

            @pl.when(b + NBUF - 1 < NBLK)
            def _():
                start_idx(b + NBUF - 1, q3)

            q2 = (p + NBUF - 2) % NBUF

            @pl.when(b + NBUF - 2 < NBLK)
            def _():
                start_fetch(b + NBUF - 2, q2)
        return carry

    lax.fori_loop(0, NBLK // NBUF, outer, 0)

    for t in range(NBLK - NBLK % NBUF, NBLK):
        p = t % NBUF
        compute(t, p)
        wait_scatter((p + NBUF - 1) % NBUF)
    wait_scatter((NBLK - 1) % NBUF)
    plsc.subcore_barrier()

    pltpu.sync_copy(agg_sh.at[pl.ds(s * RPT, RPT)],
                    out_hbm.at[c, pl.ds(s * RPT, RPT)])

    @pl.when(s == NS - 1)
    def _():
        pltpu.sync_copy(agg_sh.at[pl.ds(NS * RPT, TAIL)],
                        out_hbm.at[c, pl.ds(NS * RPT, TAIL)])


def _mlp_body(x_ref, p_ref, w1t_ref, b1_ref, gamma_ref, beta_ref,
              w2t_ref, b2_ref, scale_ref, out_ref):
    h = scale_ref[0, 0] * x_ref[...] + p_ref[0] + p_ref[1]
    h1 = jnp.dot(h, w1t_ref[...], preferred_element_type=jnp.float32)
    h1 = h1 + b1_ref[...]
    mean = jnp.mean(h1, axis=0, keepdims=True)
    cent = h1 - mean
    var = jnp.mean(cent * cent, axis=0, keepdims=True)
    hn = cent * lax.rsqrt(var + 1e-5) * gamma_ref[...] + beta_ref[...]
    hr = jnp.maximum(hn, 0.0)
    out = jnp.dot(hr, w2t_ref[...], preferred_element_type=jnp.float32)
    out_ref[...] = out + b2_ref[...]


_mlp_call = pl.pallas_call(
    _mlp_body,
    out_shape=jax.ShapeDtypeStruct((N, D), jnp.float32),
)


def kernel(x, edge_index, edge_embed, W1, b1, gamma, beta, W2, b2, eps):
    ei = edge_index.astype(jnp.int32)
    row = ei[0]
    col = ei[1]
    zero = jnp.zeros((N, D), jnp.float32)
    partials = _sc_agg(x, row, col, edge_embed, zero)
    scale = (1.0 + eps[0]).reshape(1, 1)
    return _mlp_call(x, partials, W1.T, b1.reshape(1, D),
                     gamma.reshape(1, D), beta.reshape(1, D),
                     W2.T, b2.reshape(1, D), scale)


# speedup vs baseline: 8.0458x; 1.0043x over previous
import functools

import jax
import jax.numpy as jnp
from jax import lax
from jax.experimental import pallas as pl
from jax.experimental.pallas import tpu as pltpu
from jax.experimental.pallas import tpu_sc as plsc

N = 10000
E = 320000
D = 128

NC = 2
NS = 16
NW = NC * NS
EPW = E // NW
K = 40
NBLK = EPW // K
NBUF = 4
RPT = 624
TAIL = N - NS * RPT

_mesh = plsc.VectorSubcoreMesh(core_axis_name="c", subcore_axis_name="s")


@functools.partial(
    pl.kernel,
    mesh=_mesh,
    out_type=jax.ShapeDtypeStruct((NC, N, D), jnp.float32),
    scratch_types=(
        [pltpu.VMEM_SHARED((N, D), jnp.float32)]
        + [pltpu.VMEM((K,), jnp.int32)] * NBUF
        + [pltpu.VMEM((K,), jnp.int32)] * NBUF
        + [pltpu.VMEM((K, D), jnp.float32)] * NBUF
        + [pltpu.VMEM((K, D), jnp.float32)] * NBUF
        + [pltpu.SemaphoreType.DMA] * (5 * NBUF)
    ),
)
def _sc_agg(x_hbm, row_hbm, col_hbm, ee_hbm, out_hbm,
            agg_sh, *bufs):
    colv = bufs[0:NBUF]
    rowv = bufs[NBUF:2 * NBUF]
    xg = bufs[2 * NBUF:3 * NBUF]
    eev = bufs[3 * NBUF:4 * NBUF]
    csem = bufs[4 * NBUF:5 * NBUF]
    rsem = bufs[5 * NBUF:6 * NBUF]
    gsem = bufs[6 * NBUF:7 * NBUF]
    esem = bufs[7 * NBUF:8 * NBUF]
    ssem = bufs[8 * NBUF:9 * NBUF]

    c = lax.axis_index("c")
    s = lax.axis_index("s")
    wid = c * NS + s
    base = wid * EPW

    z = jnp.zeros((16,), jnp.float32)

    def zrows(i, carry):
        for j in range(D // 16):
            xg[0][i, pl.ds(j * 16, 16)] = z
        return carry

    lax.fori_loop(0, K, zrows, 0)
    for i in range(RPT // K):
        pltpu.sync_copy(xg[0], agg_sh.at[pl.ds(s * RPT + i * K, K)])
    rem = RPT - (RPT // K) * K
    if rem:
        pltpu.sync_copy(xg[0].at[pl.ds(0, rem)],
                        agg_sh.at[pl.ds(s * RPT + (RPT // K) * K, rem)])

    @pl.when(s == NS - 1)
    def _():
        pltpu.sync_copy(xg[0].at[pl.ds(0, TAIL)],
                        agg_sh.at[pl.ds(NS * RPT, TAIL)])

    plsc.subcore_barrier()

    def start_idx(n, q):
        off = base + n * K
        pltpu.async_copy(col_hbm.at[pl.ds(off, K)], colv[q], csem[q])
        pltpu.async_copy(row_hbm.at[pl.ds(off, K)], rowv[q], rsem[q])

    def start_fetch(n, q):
        pltpu.make_async_copy(col_hbm.at[pl.ds(0, K)], colv[q],
                              csem[q]).wait()
        pltpu.async_copy(x_hbm.at[colv[q]], xg[q], gsem[q])
        pltpu.async_copy(ee_hbm.at[pl.ds(base + n * K, K)], eev[q], esem[q])

    def wait_scatter(q):
        pltpu.make_async_copy(xg[q], agg_sh.at[rowv[q]], ssem[q]).wait()

    def compute(b, p):
        pltpu.make_async_copy(x_hbm.at[colv[p]], xg[p], gsem[p]).wait()
        pltpu.make_async_copy(ee_hbm.at[pl.ds(0, K)], eev[p], esem[p]).wait()

        def mrows(i, carry):
            r = i * 8
            for rr in range(8):
                for j in range(D // 16):
                    sl = pl.ds(j * 16, 16)
                    xg[p][r + rr, sl] = xg[p][r + rr, sl] * eev[p][r + rr, sl]
            return carry

        lax.fori_loop(0, K // 8, mrows, 0)
        pltpu.make_async_copy(row_hbm.at[pl.ds(0, K)], rowv[p],
                              rsem[p]).wait()
        pltpu.async_copy(xg[p], agg_sh.at[rowv[p]], ssem[p], add=True)

    for n in range(NBUF - 1):
        start_idx(n, n)
    for n in range(NBUF - 2):
        start_fetch(n, n)

    def outer(g, carry):
        for p in range(NBUF):
            b = g * NBUF + p
            compute(b, p)
            q3 = (p + NBUF - 1) % NBUF

            @pl.when(b >= 1)
            def _():
                wait_scatter(q3)

            @pl.when(b + NBUF - 1 < NBLK)
            def _():
                start_idx(b + NBUF - 1, q3)

            q2 = (p + NBUF - 2) % NBUF

            @pl.when(b + NBUF - 2 < NBLK)
            def _():
                start_fetch(b + NBUF - 2, q2)
        return carry

    lax.fori_loop(0, NBLK // NBUF, outer, 0)

    for t in range(NBLK - NBLK % NBUF, NBLK):
        p = t % NBUF
        compute(t, p)
        wait_scatter((p + NBUF - 1) % NBUF)
    wait_scatter((NBLK - 1) % NBUF)
    plsc.subcore_barrier()

    pltpu.sync_copy(agg_sh.at[pl.ds(s * RPT, RPT)],
                    out_hbm.at[c, pl.ds(s * RPT, RPT)])

    @pl.when(s == NS - 1)
    def _():
        pltpu.sync_copy(agg_sh.at[pl.ds(NS * RPT, TAIL)],
                        out_hbm.at[c, pl.ds(NS * RPT, TAIL)])


def _mlp_body(x_ref, p_ref, w1t_ref, b1_ref, gamma_ref, beta_ref,
              w2t_ref, b2_ref, scale_ref, out_ref):
    h = scale_ref[0, 0] * x_ref[...] + p_ref[0] + p_ref[1]
    h1 = jnp.dot(h, w1t_ref[...], preferred_element_type=jnp.float32)
    h1 = h1 + b1_ref[...]
    mean = jnp.mean(h1, axis=0, keepdims=True)
    cent = h1 - mean
    var = jnp.mean(cent * cent, axis=0, keepdims=True)
    hn = cent * lax.rsqrt(var + 1e-5) * gamma_ref[...] + beta_ref[...]
    hr = jnp.maximum(hn, 0.0)
    out = jnp.dot(hr, w2t_ref[...], preferred_element_type=jnp.float32)
    out_ref[...] = out + b2_ref[...]


_mlp_call = pl.pallas_call(
    _mlp_body,
    out_shape=jax.ShapeDtypeStruct((N, D), jnp.float32),
)


def kernel(x, edge_index, edge_embed, W1, b1, gamma, beta, W2, b2, eps):
    ei = edge_index.astype(jnp.int32)
    row = ei[0]
    col = ei[1]
    partials = _sc_agg(x, row, col, edge_embed)
    scale = (1.0 + eps[0]).reshape(1, 1)
    return _mlp_call(x, partials, W1.T, b1.reshape(1, D),
                     gamma.reshape(1, D), beta.reshape(1, D),
                     W2.T, b2.reshape(1, D), scale)


# speedup vs baseline: 8.0867x; 1.0051x over previous
import functools

import jax
import jax.numpy as jnp
from jax import lax
from jax.experimental import pallas as pl
from jax.experimental.pallas import tpu as pltpu
from jax.experimental.pallas import tpu_sc as plsc

N = 10000
E = 320000
D = 128

NC = 2
NS = 16
NW = NC * NS
EPW = E // NW
K = 40
NBLK = EPW // K
NBUF = 4
RPT = 624
TAIL = N - NS * RPT

_mesh = plsc.VectorSubcoreMesh(core_axis_name="c", subcore_axis_name="s")


@functools.partial(
    pl.kernel,
    mesh=_mesh,
    out_type=jax.ShapeDtypeStruct((NC, N, D), jnp.float32),
    scratch_types=(
        [pltpu.VMEM_SHARED((N, D), jnp.float32)]
        + [pltpu.VMEM((K,), jnp.int32)] * NBUF
        + [pltpu.VMEM((K,), jnp.int32)] * NBUF
        + [pltpu.VMEM((K, D), jnp.float32)] * NBUF
        + [pltpu.VMEM((K, D), jnp.float32)] * NBUF
        + [pltpu.SemaphoreType.DMA] * (5 * NBUF)
    ),
)
def _sc_agg(x_hbm, row_hbm, col_hbm, ee_hbm, out_hbm,
            agg_sh, *bufs):
    colv = bufs[0:NBUF]
    rowv = bufs[NBUF:2 * NBUF]
    xg = bufs[2 * NBUF:3 * NBUF]
    eev = bufs[3 * NBUF:4 * NBUF]
    csem = bufs[4 * NBUF:5 * NBUF]
    rsem = bufs[5 * NBUF:6 * NBUF]
    gsem = bufs[6 * NBUF:7 * NBUF]
    esem = bufs[7 * NBUF:8 * NBUF]
    ssem = bufs[8 * NBUF:9 * NBUF]

    c = lax.axis_index("c")
    s = lax.axis_index("s")
    wid = c * NS + s
    base = wid * EPW

    z = jnp.zeros((16,), jnp.float32)

    def zrows(i, carry):
        for j in range(D // 16):
            xg[0][i, pl.ds(j * 16, 16)] = z
        return carry

    lax.fori_loop(0, K, zrows, 0)
    rem = RPT - (RPT // K) * K
    for i in range(RPT // K):
        pltpu.async_copy(xg[0], agg_sh.at[pl.ds(s * RPT + i * K, K)],
                         csem[0])
    pltpu.async_copy(xg[0].at[pl.ds(0, rem)],
                     agg_sh.at[pl.ds(s * RPT + (RPT // K) * K, rem)],
                     csem[0])

    @pl.when(s == NS - 1)
    def _():
        pltpu.async_copy(xg[0].at[pl.ds(0, TAIL)],
                         agg_sh.at[pl.ds(NS * RPT, TAIL)], csem[0])
        pltpu.make_async_copy(xg[0].at[pl.ds(0, TAIL)],
                              agg_sh.at[pl.ds(NS * RPT, TAIL)],
                              csem[0]).wait()

    for i in range(RPT // K):
        pltpu.make_async_copy(xg[0], agg_sh.at[pl.ds(0, K)],
                              csem[0]).wait()
    pltpu.make_async_copy(xg[0].at[pl.ds(0, rem)],
                          agg_sh.at[pl.ds(0, rem)], csem[0]).wait()

    plsc.subcore_barrier()

    def start_idx(n, q):
        off = base + n * K
        pltpu.async_copy(col_hbm.at[pl.ds(off, K)], colv[q], csem[q])
        pltpu.async_copy(row_hbm.at[pl.ds(off, K)], rowv[q], rsem[q])

    def start_fetch(n, q):
        pltpu.make_async_copy(col_hbm.at[pl.ds(0, K)], colv[q],
                              csem[q]).wait()
        pltpu.async_copy(x_hbm.at[colv[q]], xg[q], gsem[q])
        pltpu.async_copy(ee_hbm.at[pl.ds(base + n * K, K)], eev[q], esem[q])

    def wait_scatter(q):
        pltpu.make_async_copy(xg[q], agg_sh.at[rowv[q]], ssem[q]).wait()

    def compute(b, p):
        pltpu.make_async_copy(x_hbm.at[colv[p]], xg[p], gsem[p]).wait()
        pltpu.make_async_copy(ee_hbm.at[pl.ds(0, K)], eev[p], esem[p]).wait()

        def mrows(i, carry):
            r = i * 8
            for rr in range(8):
                for j in range(D // 16):
                    sl = pl.ds(j * 16, 16)
                    xg[p][r + rr, sl] = xg[p][r + rr, sl] * eev[p][r + rr, sl]
            return carry

        lax.fori_loop(0, K // 8, mrows, 0)
        pltpu.make_async_copy(row_hbm.at[pl.ds(0, K)], rowv[p],
                              rsem[p]).wait()
        pltpu.async_copy(xg[p], agg_sh.at[rowv[p]], ssem[p], add=True)

    for n in range(NBUF - 1):
        start_idx(n, n)
    for n in range(NBUF - 2):
        start_fetch(n, n)

    def outer(g, carry):
        for p in range(NBUF):
            b = g * NBUF + p
            compute(b, p)
            q3 = (p + NBUF - 1) % NBUF

            @pl.when(b >= 1)
            def _():
                wait_scatter(q3)

            @pl.when(b + NBUF - 1 < NBLK)
            def _():
                start_idx(b + NBUF - 1, q3)

            q2 = (p + NBUF - 2) % NBUF

            @pl.when(b + NBUF - 2 < NBLK)
            def _():
                start_fetch(b + NBUF - 2, q2)
        return carry

    lax.fori_loop(0, NBLK // NBUF, outer, 0)

    for t in range(NBLK - NBLK % NBUF, NBLK):
        p = t % NBUF
        compute(t, p)
        wait_scatter((p + NBUF - 1) % NBUF)
    wait_scatter((NBLK - 1) % NBUF)
    plsc.subcore_barrier()

    pltpu.sync_copy(agg_sh.at[pl.ds(s * RPT, RPT)],
                    out_hbm.at[c, pl.ds(s * RPT, RPT)])

    @pl.when(s == NS - 1)
    def _():
        pltpu.sync_copy(agg_sh.at[pl.ds(NS * RPT, TAIL)],
                        out_hbm.at[c, pl.ds(NS * RPT, TAIL)])


def _mlp_body(x_ref, p_ref, w1t_ref, b1_ref, gamma_ref, beta_ref,
              w2t_ref, b2_ref, scale_ref, out_ref):
    h = scale_ref[0, 0] * x_ref[...] + p_ref[0] + p_ref[1]
    h1 = jnp.dot(h, w1t_ref[...], preferred_element_type=jnp.float32)
    h1 = h1 + b1_ref[...]
    mean = jnp.mean(h1, axis=0, keepdims=True)
    cent = h1 - mean
    var = jnp.mean(cent * cent, axis=0, keepdims=True)
    hn = cent * lax.rsqrt(var + 1e-5) * gamma_ref[...] + beta_ref[...]
    hr = jnp.maximum(hn, 0.0)
    out = jnp.dot(hr, w2t_ref[...], preferred_element_type=jnp.float32)
    out_ref[...] = out + b2_ref[...]


_mlp_call = pl.pallas_call(
    _mlp_body,
    out_shape=jax.ShapeDtypeStruct((N, D), jnp.float32),
)


def kernel(x, edge_index, edge_embed, W1, b1, gamma, beta, W2, b2, eps):
    ei = edge_index.astype(jnp.int32)
    row = ei[0]
    col = ei[1]
    partials = _sc_agg(x, row, col, edge_embed)
    scale = (1.0 + eps[0]).reshape(1, 1)
    return _mlp_call(x, partials, W1.T, b1.reshape(1, D),
                     gamma.reshape(1, D), beta.reshape(1, D),
                     W2.T, b2.reshape(1, D), scale)
